# baseline (speedup 1.0000x reference)
import functools

import jax
import jax.numpy as jnp
from jax import lax
from jax.experimental import pallas as pl
from jax.experimental.pallas import tpu as pltpu
from jax.experimental.pallas import tpu_sc as plsc

N = 10000
E = 320000
H = 128
NRANGE = 64
RSZ = 160
NP = NRANGE * RSZ
NWORK = 32
CAP = 6144
CAP16 = CAP + 16
ECH = 6400
NCH = E // ECH
GCH = 64
NEG = -3.0e38
POS = 3.0e38


def _node_in_body(x_ref, w_ref, b_ref, wd_ref, ws_ref, h_ref, a_ref, b2_ref):
    h = jax.nn.relu(
        jnp.dot(x_ref[...], w_ref[...], preferred_element_type=jnp.float32)
        + b_ref[...])
    h_ref[...] = h
    a_ref[...] = jnp.dot(h, wd_ref[...], preferred_element_type=jnp.float32)
    b2_ref[...] = jnp.dot(h, ws_ref[...], preferred_element_type=jnp.float32)


def _node_in(x, node_W, node_b, Wd, Ws):
    blk = 1000
    full = lambda i: (0, 0)
    return pl.pallas_call(
        _node_in_body,
        grid=(N // blk,),
        in_specs=[
            pl.BlockSpec((blk, H), lambda i: (i, 0)),
            pl.BlockSpec((H, H), full),
            pl.BlockSpec((H,), lambda i: (0,)),
            pl.BlockSpec((H, H), full),
            pl.BlockSpec((H, H), full),
        ],
        out_specs=[pl.BlockSpec((blk, H), lambda i: (i, 0))] * 3,
        out_shape=[jax.ShapeDtypeStruct((N, H), jnp.float32)] * 3,
    )(x, node_W, node_b, Wd, Ws)


def _edges_body(ea_ref, we_ref, be_ref, w1_ref, b1_ref, w2_ref, b2_ref,
                c1_ref, c2_ref):
    e = jax.nn.relu(
        jnp.dot(ea_ref[...], we_ref[...], preferred_element_type=jnp.float32)
        + be_ref[...])
    c1_ref[...] = jnp.dot(e, w1_ref[...], preferred_element_type=jnp.float32) + b1_ref[...]
    c2_ref[...] = jnp.dot(e, w2_ref[...], preferred_element_type=jnp.float32) + b2_ref[...]


def _edges(edge_attr, edge_W, edge_b, We1, pre1_b, We2, pre2_b):
    blk = 2000
    full = lambda i: (0, 0)
    return pl.pallas_call(
        _edges_body,
        grid=(E // blk,),
        in_specs=[
            pl.BlockSpec((blk, 16), lambda i: (i, 0)),
            pl.BlockSpec((16, H), full),
            pl.BlockSpec((H,), lambda i: (0,)),
            pl.BlockSpec((H, H), full),
            pl.BlockSpec((H,), lambda i: (0,)),
            pl.BlockSpec((H, H), full),
            pl.BlockSpec((H,), lambda i: (0,)),
        ],
        out_specs=[pl.BlockSpec((blk, H), lambda i: (i, 0))] * 2,
        out_shape=[jax.ShapeDtypeStruct((E, H), jnp.float32)] * 2,
    )(edge_attr, edge_W, edge_b, We1, pre1_b, We2, pre2_b)


def _post_body(h_ref, a_ref, s_ref, q_ref, mx_ref, mn_ref, cnt_ref,
               pw_ref, pb_ref, lw_ref, lb_ref, g_ref, bt_ref,
               wdn_ref, wsn_ref, hn_ref, an_ref, bn_ref):
    h = h_ref[...]
    A = a_ref[...]
    cnt = cnt_ref[...][:, 0:1]
    invd = 1.0 / jnp.maximum(cnt, 1.0)
    mean_t = s_ref[...] * invd
    var = jax.nn.relu(q_ref[...] * invd - mean_t * mean_t)
    std = jnp.sqrt(var + 1e-5)
    mask = cnt > 0.0
    mean = jnp.where(mask, A + mean_t, 0.0)
    mn = jnp.where(mask, A + mn_ref[...], 0.0)
    mx = jnp.where(mask, A + mx_ref[...], 0.0)
    pw = pw_ref[...]
    dot = lambda a, b: jnp.dot(a, b, preferred_element_type=jnp.float32)
    out = (dot(h, pw[0:H]) + dot(mean, pw[H:2 * H]) + dot(mn, pw[2 * H:3 * H])
           + dot(mx, pw[3 * H:4 * H]) + dot(std, pw[4 * H:5 * H]) + pb_ref[...])
    out = dot(out, lw_ref[...]) + lb_ref[...]
    y = jnp.where(out >= 0.0, out, 0.01 * out)
    mu = jnp.mean(y, axis=-1, keepdims=True)
    v = jnp.mean((y - mu) ** 2, axis=-1, keepdims=True)
    hn = (y - mu) / jnp.sqrt(v + 1e-5) * g_ref[...] + bt_ref[...]
    hn_ref[...] = hn
    an_ref[...] = dot(hn, wdn_ref[...])
    bn_ref[...] = dot(hn, wsn_ref[...])


def _post(h, A, S, Q, Mx, Mn, CNT, post_W, post_b, lin_W, lin_b, g, bt,
          Wdn, Wsn):
    blk = 1000
    full = lambda i: (0, 0)
    vec = lambda i: (0,)
    return pl.pallas_call(
        _post_body,
        grid=(N // blk,),
        in_specs=[
            pl.BlockSpec((blk, H), lambda i: (i, 0)),
            pl.BlockSpec((blk, H), lambda i: (i, 0)),
            pl.BlockSpec((blk, H), lambda i: (i, 0)),
            pl.BlockSpec((blk, H), lambda i: (i, 0)),
            pl.BlockSpec((blk, H), lambda i: (i, 0)),
            pl.BlockSpec((blk, H), lambda i: (i, 0)),
            pl.BlockSpec((blk, 16), lambda i: (i, 0)),
            pl.BlockSpec((5 * H, H), full),
            pl.BlockSpec((H,), vec),
            pl.BlockSpec((H, H), full),
            pl.BlockSpec((H,), vec),
            pl.BlockSpec((H,), vec),
            pl.BlockSpec((H,), vec),
            pl.BlockSpec((H, H), full),
            pl.BlockSpec((H, H), full),
        ],
        out_specs=[pl.BlockSpec((blk, H), lambda i: (i, 0))] * 3,
        out_shape=[jax.ShapeDtypeStruct((N, H), jnp.float32)] * 3,
    )(h, A, S, Q, Mx, Mn, CNT, post_W, post_b, lin_W, lin_b, g, bt, Wdn, Wsn)


def _sc_mesh():
    return plsc.VectorSubcoreMesh(core_axis_name="c", subcore_axis_name="s")


def _prep_body(src_hbm, dst_hbm, eids_hbm, srcs_hbm, dsts_hbm, cnts_hbm,
               dstbuf, srcbuf, el0, sl0, dl0, el1, sl1, dl1, crow):
    w = lax.axis_index("c") * 16 + lax.axis_index("s")
    lo = w * (2 * RSZ)
    mid = lo + RSZ
    hi = lo + 2 * RSZ

    def zero(i, _):
        z = jnp.zeros((16,), jnp.int32)
        sl = pl.ds(i * 16, 16)
        el0[sl] = z
        sl0[sl] = z
        dl0[sl] = z
        el1[sl] = z
        sl1[sl] = z
        dl1[sl] = z
        return 0
    lax.fori_loop(0, CAP16 // 16, zero, 0)

    ONES = jnp.full((16,), 1, jnp.int32)
    ZEROS = jnp.full((16,), 0, jnp.int32)

    def chunk(ci, offs):
        pltpu.sync_copy(dst_hbm.at[pl.ds(ci * ECH, ECH)], dstbuf)
        pltpu.sync_copy(src_hbm.at[pl.ds(ci * ECH, ECH)], srcbuf)

        def step(si, offs):
            o0, o1 = offs
            sl = pl.ds(si * 16, 16)
            d16 = dstbuf[sl]
            s16 = srcbuf[sl]
            e16 = ci * ECH + si * 16 + lax.iota(jnp.int32, 16)
            m0 = (d16 >= lo) & (d16 < mid)
            m1 = (d16 >= mid) & (d16 < hi)
            n0 = plsc.all_reduce_population_count(m0)[0]
            n1 = plsc.all_reduce_population_count(m1)[0]

            @pl.when(n0 > 0)
            def _():
                plsc.store_compressed(el0.at[pl.ds(o0, 16)], e16, mask=m0)
                plsc.store_compressed(sl0.at[pl.ds(o0, 16)], s16, mask=m0)
                plsc.store_compressed(dl0.at[pl.ds(o0, 16)], d16, mask=m0)

            @pl.when(n1 > 0)
            def _():
                plsc.store_compressed(el1.at[pl.ds(o1, 16)], e16, mask=m1)
                plsc.store_compressed(sl1.at[pl.ds(o1, 16)], s16, mask=m1)
                plsc.store_compressed(dl1.at[pl.ds(o1, 16)], d16, mask=m1)

            return (jnp.minimum(o0 + n0, CAP), jnp.minimum(o1 + n1, CAP))

        return lax.fori_loop(0, ECH // 16, step, offs)

    o0, o1 = lax.fori_loop(0, NCH, chunk, (0, 0))

    r0 = 2 * w
    r1 = 2 * w + 1
    pltpu.sync_copy(el0.at[pl.ds(0, CAP)], eids_hbm.at[r0])
    pltpu.sync_copy(sl0.at[pl.ds(0, CAP)], srcs_hbm.at[r0])
    pltpu.sync_copy(dl0.at[pl.ds(0, CAP)], dsts_hbm.at[r0])
    pltpu.sync_copy(el1.at[pl.ds(0, CAP)], eids_hbm.at[r1])
    pltpu.sync_copy(sl1.at[pl.ds(0, CAP)], srcs_hbm.at[r1])
    pltpu.sync_copy(dl1.at[pl.ds(0, CAP)], dsts_hbm.at[r1])
    crow[...] = jnp.full((16,), o0, jnp.int32)
    pltpu.sync_copy(crow, cnts_hbm.at[r0])
    crow[...] = jnp.full((16,), o1, jnp.int32)
    pltpu.sync_copy(crow, cnts_hbm.at[r1])


def _sc_prep(src, dst):
    ilist = jax.ShapeDtypeStruct((NRANGE, CAP), jnp.int32)
    k = pl.kernel(
        _prep_body,
        compiler_params=pltpu.CompilerParams(needs_layout_passes=False),
        out_type=[ilist, ilist, ilist,
                  jax.ShapeDtypeStruct((NRANGE, 16), jnp.int32)],
        mesh=_sc_mesh(),
        scratch_types=[
            pltpu.VMEM((ECH,), jnp.int32),
            pltpu.VMEM((ECH,), jnp.int32),
            pltpu.VMEM((CAP16,), jnp.int32),
            pltpu.VMEM((CAP16,), jnp.int32),
            pltpu.VMEM((CAP16,), jnp.int32),
            pltpu.VMEM((CAP16,), jnp.int32),
            pltpu.VMEM((CAP16,), jnp.int32),
            pltpu.VMEM((CAP16,), jnp.int32),
            pltpu.VMEM((16,), jnp.int32),
        ],
    )
    return k(src, dst)


def _stats_body(b_hbm, c_hbm, eids_hbm, srcs_hbm, dsts_hbm, cnts_hbm,
                s_hbm, q_hbm, mx_hbm, mn_hbm, cnt_hbm,
                eidbuf, srcbuf, dstbuf, cbuf, bbuf,
                accS, accQ, accMx, accMn, accC, crow, sem1, sem2):
    w = lax.axis_index("c") * 16 + lax.axis_index("s")

    def do_range(rr, _):
        r = 2 * w + rr
        lo = r * RSZ

        def zero(i, _):
            zf = jnp.zeros((16,), jnp.float32)
            for k in range(8):
                sl = pl.ds(k * 16, 16)
                accS[i, sl] = zf
                accQ[i, sl] = zf
                accMx[i, sl] = jnp.full((16,), NEG, jnp.float32)
                accMn[i, sl] = jnp.full((16,), POS, jnp.float32)
            accC[i, :] = zf
            return 0
        lax.fori_loop(0, RSZ + 1, zero, 0)

        pltpu.sync_copy(cnts_hbm.at[r], crow)
        cnt = jnp.max(crow[...])
        nch = (cnt + (GCH - 1)) // GCH

        def chunk(g, _):
            base = g * GCH
            pltpu.sync_copy(eids_hbm.at[r, pl.ds(base, GCH)], eidbuf)
            pltpu.sync_copy(srcs_hbm.at[r, pl.ds(base, GCH)], srcbuf)
            pltpu.sync_copy(dsts_hbm.at[r, pl.ds(base, GCH)], dstbuf)
            cp1 = pltpu.async_copy(c_hbm.at[eidbuf], cbuf, sem1)
            cp2 = pltpu.async_copy(b_hbm.at[srcbuf], bbuf, sem2)
            cp1.wait()
            cp2.wait()
            nin = jnp.minimum(cnt - base, GCH)

            def group(gi, _):
                jb = gi * 16
                dl16 = dstbuf[pl.ds(jb, 16)] - lo
                for u in range(16):
                    j = jb + u
                    d = jnp.where(j < nin, dl16[u], RSZ)
                    for k in range(8):
                        sl = pl.ds(k * 16, 16)
                        t = bbuf[j, sl] + cbuf[j, sl]
                        accS[d, sl] = accS[d, sl] + t
                        accQ[d, sl] = accQ[d, sl] + t * t
                        accMx[d, sl] = jnp.maximum(accMx[d, sl], t)
                        accMn[d, sl] = jnp.minimum(accMn[d, sl], t)
                    accC[d, :] = accC[d, :] + 1.0
                return 0
            lax.fori_loop(0, GCH // 16, group, 0)
            return 0
        lax.fori_loop(0, nch, chunk, 0)

        pltpu.sync_copy(accS.at[pl.ds(0, RSZ)], s_hbm.at[pl.ds(lo, RSZ)])
        pltpu.sync_copy(accQ.at[pl.ds(0, RSZ)], q_hbm.at[pl.ds(lo, RSZ)])
        pltpu.sync_copy(accMx.at[pl.ds(0, RSZ)], mx_hbm.at[pl.ds(lo, RSZ)])
        pltpu.sync_copy(accMn.at[pl.ds(0, RSZ)], mn_hbm.at[pl.ds(lo, RSZ)])
        pltpu.sync_copy(accC.at[pl.ds(0, RSZ)], cnt_hbm.at[pl.ds(lo, RSZ)])
        return 0

    lax.fori_loop(0, 2, do_range, 0)


def _sc_stats(B, C, eids, srcs, dsts, cnts):
    stat = jax.ShapeDtypeStruct((NP, H), jnp.float32)
    k = pl.kernel(
        _stats_body,
        compiler_params=pltpu.CompilerParams(needs_layout_passes=False),
        out_type=[stat] * 4 + [jax.ShapeDtypeStruct((NP, 16), jnp.float32)],
        mesh=_sc_mesh(),
        scratch_types=[
            pltpu.VMEM((GCH,), jnp.int32),
            pltpu.VMEM((GCH,), jnp.int32),
            pltpu.VMEM((GCH,), jnp.int32),
            pltpu.VMEM((GCH, H), jnp.float32),
            pltpu.VMEM((GCH, H), jnp.float32),
            pltpu.VMEM((RSZ + 1, H), jnp.float32),
            pltpu.VMEM((RSZ + 1, H), jnp.float32),
            pltpu.VMEM((RSZ + 1, H), jnp.float32),
            pltpu.VMEM((RSZ + 1, H), jnp.float32),
            pltpu.VMEM((RSZ + 1, 16), jnp.float32),
            pltpu.VMEM((16,), jnp.int32),
            pltpu.SemaphoreType.DMA,
            pltpu.SemaphoreType.DMA,
        ],
    )
    return k(B, C, eids, srcs, dsts, cnts)


def kernel(x, edge_index, edge_attr, node_W, node_b, edge_W, edge_b,
           pre1_W, pre1_b, post1_W, post1_b, lin1_W, lin1_b, ln1_g, ln1_bt,
           pre2_W, pre2_b, post2_W, post2_b, lin2_W, lin2_b, ln2_g, ln2_bt):
    src = edge_index[0]
    dst = edge_index[1]
    Wd1, Ws1, We1 = pre1_W[:H], pre1_W[H:2 * H], pre1_W[2 * H:]
    Wd2, Ws2, We2 = pre2_W[:H], pre2_W[H:2 * H], pre2_W[2 * H:]

    h, A1, B1 = _node_in(x, node_W, node_b, Wd1, Ws1)
    C1, C2 = _edges(edge_attr, edge_W, edge_b, We1, pre1_b, We2, pre2_b)
    eids, srcs, dsts, cnts = _sc_prep(src, dst)

    S1, Q1, Mx1, Mn1, CNT1 = _sc_stats(B1, C1, eids, srcs, dsts, cnts)
    h1, A2, B2 = _post(h, A1, S1[:N], Q1[:N], Mx1[:N], Mn1[:N], CNT1[:N],
                       post1_W, post1_b, lin1_W, lin1_b, ln1_g, ln1_bt,
                       Wd2, Ws2)

    S2, Q2, Mx2, Mn2, CNT2 = _sc_stats(B2, C2, eids, srcs, dsts, cnts)
    h2, _, _ = _post(h1, A2, S2[:N], Q2[:N], Mx2[:N], Mn2[:N], CNT2[:N],
                     post2_W, post2_b, lin2_W, lin2_b, ln2_g, ln2_bt,
                     Wd2, Ws2)
    return h2


# speedup vs baseline: 1.5598x; 1.5598x over previous
import jax
import jax.numpy as jnp
from jax import lax
from jax.experimental import pallas as pl
from jax.experimental.pallas import tpu as pltpu
from jax.experimental.pallas import tpu_sc as plsc

N = 10000
E = 320000
H = 128
NRANGE = 64
RSZ = 160
NP = NRANGE * RSZ
CAP = 6144
CAP16 = CAP + 16
ECH = 6400
NCH = E // ECH
GCH = 64
HB = 176
NEG = -3.0e38
POS = 3.0e38


def _node_in_body(x_ref, w_ref, b_ref, wd_ref, ws_ref, h_ref, a_ref, b2_ref):
    h = jax.nn.relu(
        jnp.dot(x_ref[...], w_ref[...], preferred_element_type=jnp.float32)
        + b_ref[...])
    h_ref[...] = h
    a_ref[...] = jnp.dot(h, wd_ref[...], preferred_element_type=jnp.float32)
    b2_ref[...] = jnp.dot(h, ws_ref[...], preferred_element_type=jnp.float32)


def _node_in(x, node_W, node_b, Wd, Ws):
    blk = 1000
    full = lambda i: (0, 0)
    return pl.pallas_call(
        _node_in_body,
        grid=(N // blk,),
        in_specs=[
            pl.BlockSpec((blk, H), lambda i: (i, 0)),
            pl.BlockSpec((H, H), full),
            pl.BlockSpec((H,), lambda i: (0,)),
            pl.BlockSpec((H, H), full),
            pl.BlockSpec((H, H), full),
        ],
        out_specs=[pl.BlockSpec((blk, H), lambda i: (i, 0))] * 3,
        out_shape=[jax.ShapeDtypeStruct((N, H), jnp.float32)] * 3,
    )(x, node_W, node_b, Wd, Ws)


def _edges_body(ea_ref, we_ref, be_ref, w1_ref, b1_ref, w2_ref, b2_ref,
                c1_ref, c2_ref):
    e = jax.nn.relu(
        jnp.dot(ea_ref[...], we_ref[...], preferred_element_type=jnp.float32)
        + be_ref[...])
    c1_ref[...] = jnp.dot(e, w1_ref[...], preferred_element_type=jnp.float32) + b1_ref[...]
    c2_ref[...] = jnp.dot(e, w2_ref[...], preferred_element_type=jnp.float32) + b2_ref[...]


def _edges(edge_attr, edge_W, edge_b, We1, pre1_b, We2, pre2_b):
    blk = 2000
    full = lambda i: (0, 0)
    return pl.pallas_call(
        _edges_body,
        grid=(E // blk,),
        in_specs=[
            pl.BlockSpec((blk, 16), lambda i: (i, 0)),
            pl.BlockSpec((16, H), full),
            pl.BlockSpec((H,), lambda i: (0,)),
            pl.BlockSpec((H, H), full),
            pl.BlockSpec((H,), lambda i: (0,)),
            pl.BlockSpec((H, H), full),
            pl.BlockSpec((H,), lambda i: (0,)),
        ],
        out_specs=[pl.BlockSpec((blk, H), lambda i: (i, 0))] * 2,
        out_shape=[jax.ShapeDtypeStruct((E, H), jnp.float32)] * 2,
    )(edge_attr, edge_W, edge_b, We1, pre1_b, We2, pre2_b)


def _post_body(h_ref, a_ref, s_ref, q_ref, mx_ref, mn_ref, cnt_ref,
               pw_ref, pb_ref, lw_ref, lb_ref, g_ref, bt_ref,
               wdn_ref, wsn_ref, hn_ref, an_ref, bn_ref):
    h = h_ref[...]
    A = a_ref[...]
    cnt = cnt_ref[...][:, 0:1]
    invd = 1.0 / jnp.maximum(cnt, 1.0)
    mean_t = s_ref[...] * invd
    var = jax.nn.relu(q_ref[...] * invd - mean_t * mean_t)
    std = jnp.sqrt(var + 1e-5)
    mask = cnt > 0.0
    mean = jnp.where(mask, A + mean_t, 0.0)
    mn = jnp.where(mask, A + mn_ref[...], 0.0)
    mx = jnp.where(mask, A + mx_ref[...], 0.0)
    pw = pw_ref[...]
    dot = lambda a, b: jnp.dot(a, b, preferred_element_type=jnp.float32)
    out = (dot(h, pw[0:H]) + dot(mean, pw[H:2 * H]) + dot(mn, pw[2 * H:3 * H])
           + dot(mx, pw[3 * H:4 * H]) + dot(std, pw[4 * H:5 * H]) + pb_ref[...])
    out = dot(out, lw_ref[...]) + lb_ref[...]
    y = jnp.where(out >= 0.0, out, 0.01 * out)
    mu = jnp.mean(y, axis=-1, keepdims=True)
    v = jnp.mean((y - mu) ** 2, axis=-1, keepdims=True)
    hn = (y - mu) / jnp.sqrt(v + 1e-5) * g_ref[...] + bt_ref[...]
    hn_ref[...] = hn
    an_ref[...] = dot(hn, wdn_ref[...])
    bn_ref[...] = dot(hn, wsn_ref[...])


def _post(h, A, S, Q, Mx, Mn, CNT, post_W, post_b, lin_W, lin_b, g, bt,
          Wdn, Wsn):
    blk = 1000
    full = lambda i: (0, 0)
    vec = lambda i: (0,)
    return pl.pallas_call(
        _post_body,
        grid=(N // blk,),
        in_specs=[
            pl.BlockSpec((blk, H), lambda i: (i, 0)),
            pl.BlockSpec((blk, H), lambda i: (i, 0)),
            pl.BlockSpec((blk, H), lambda i: (i, 0)),
            pl.BlockSpec((blk, H), lambda i: (i, 0)),
            pl.BlockSpec((blk, H), lambda i: (i, 0)),
            pl.BlockSpec((blk, H), lambda i: (i, 0)),
            pl.BlockSpec((blk, 16), lambda i: (i, 0)),
            pl.BlockSpec((5 * H, H), full),
            pl.BlockSpec((H,), vec),
            pl.BlockSpec((H, H), full),
            pl.BlockSpec((H,), vec),
            pl.BlockSpec((H,), vec),
            pl.BlockSpec((H,), vec),
            pl.BlockSpec((H, H), full),
            pl.BlockSpec((H, H), full),
        ],
        out_specs=[pl.BlockSpec((blk, H), lambda i: (i, 0))] * 3,
        out_shape=[jax.ShapeDtypeStruct((N, H), jnp.float32)] * 3,
    )(h, A, S, Q, Mx, Mn, CNT, post_W, post_b, lin_W, lin_b, g, bt, Wdn, Wsn)


def _sc_mesh():
    return plsc.VectorSubcoreMesh(core_axis_name="c", subcore_axis_name="s")


def _prep_body(src_hbm, dst_hbm, eids_hbm, srcs_hbm, dsts_hbm, cnts_hbm,
               dstbuf, srcbuf, el0, sl0, dl0, el1, sl1, dl1,
               sel, ssl, sdl, hist, off, tmpa, tmpb, crow):
    w = lax.axis_index("c") * 16 + lax.axis_index("s")
    lo = w * (2 * RSZ)
    mid = lo + RSZ
    hi = lo + 2 * RSZ
    IOTA = lax.iota(jnp.int32, 16)
    ZER = jnp.zeros((16,), jnp.int32)
    ONE = jnp.full((16,), 1, jnp.int32)

    def zero(i, _):
        z = jnp.zeros((16,), jnp.int32)
        sl = pl.ds(i * 16, 16)
        el0[sl] = z
        sl0[sl] = z
        dl0[sl] = z
        el1[sl] = z
        sl1[sl] = z
        dl1[sl] = z
        sel[sl] = z
        ssl[sl] = z
        sdl[sl] = z
        return 0
    lax.fori_loop(0, CAP16 // 16, zero, 0)

    def chunk(ci, offs):
        pltpu.sync_copy(dst_hbm.at[pl.ds(ci * ECH, ECH)], dstbuf)
        pltpu.sync_copy(src_hbm.at[pl.ds(ci * ECH, ECH)], srcbuf)

        def step(si, offs):
            o0, o1 = offs
            sl = pl.ds(si * 16, 16)
            d16 = dstbuf[sl]
            s16 = srcbuf[sl]
            e16 = ci * ECH + si * 16 + IOTA
            m0 = (d16 >= lo) & (d16 < mid)
            m1 = (d16 >= mid) & (d16 < hi)
            n0 = plsc.all_reduce_population_count(m0)[0]
            n1 = plsc.all_reduce_population_count(m1)[0]

            @pl.when(n0 > 0)
            def _():
                plsc.store_compressed(el0.at[pl.ds(o0, 16)], e16, mask=m0)
                plsc.store_compressed(sl0.at[pl.ds(o0, 16)], s16, mask=m0)
                plsc.store_compressed(dl0.at[pl.ds(o0, 16)], d16, mask=m0)

            @pl.when(n1 > 0)
            def _():
                plsc.store_compressed(el1.at[pl.ds(o1, 16)], e16, mask=m1)
                plsc.store_compressed(sl1.at[pl.ds(o1, 16)], s16, mask=m1)
                plsc.store_compressed(dl1.at[pl.ds(o1, 16)], d16, mask=m1)

            return (jnp.minimum(o0 + n0, CAP), jnp.minimum(o1 + n1, CAP))

        return lax.fori_loop(0, ECH // 16, step, offs)

    o0, o1 = lax.fori_loop(0, NCH, chunk, (0, 0))

    tmpb[pl.ds(16, 16)] = ONE

    def sort_range(elX, slX, dlX, rlo, cnt, rrow):
        def zh(i, _):
            hist[pl.ds(i * 16, 16)] = ZER
            return 0
        lax.fori_loop(0, HB // 16, zh, 0)

        ngroups = (cnt + 15) // 16

        def keys_of(gi):
            jb = gi * 16
            draw = dlX[pl.ds(jb, 16)] - rlo
            valid = (jb + IOTA) < cnt
            d16 = jnp.where(valid, draw, jnp.full((16,), RSZ, jnp.int32))
            sd, perm = plsc.sort_key_val(d16, IOTA)
            tmpa[pl.ds(0, 16)] = jnp.full((16,), -1, jnp.int32)
            tmpa[pl.ds(1, 16)] = sd
            prev = tmpa[pl.ds(0, 16)]
            is_start = sd != prev
            runstart = plsc.cummax(jnp.where(is_start, IOTA, ZER))
            rank = IOTA - runstart
            tmpb[pl.ds(0, 16)] = jnp.where(is_start, ONE, ZER)
            is_last = tmpb[pl.ds(1, 16)] > 0
            return jb, sd, perm, rank, is_last

        def hgroup(gi, _):
            _, sd, _, rank, is_last = keys_of(gi)
            old = plsc.load_gather(hist, [sd])
            plsc.store_scatter(hist, [sd], old + rank + 1, mask=is_last)
            return 0
        lax.fori_loop(0, ngroups, hgroup, 0)

        carry = 0
        for gi in range(HB // 16):
            v = hist[pl.ds(gi * 16, 16)]
            incl = plsc.cumsum(v)
            off[pl.ds(gi * 16, 16)] = incl - v + carry
            carry = carry + incl[15]

        def pgroup(gi, _):
            jb, sd, perm, rank, is_last = keys_of(gi)
            base = plsc.load_gather(off, [sd])
            pos = base + rank
            gidx = perm + jb
            plsc.store_scatter(sel, [pos], plsc.load_gather(elX, [gidx]))
            plsc.store_scatter(ssl, [pos], plsc.load_gather(slX, [gidx]))
            plsc.store_scatter(sdl, [pos], plsc.load_gather(dlX, [gidx]))
            plsc.store_scatter(off, [sd], pos + 1, mask=is_last)
            return 0
        lax.fori_loop(0, ngroups, pgroup, 0)

        pltpu.sync_copy(sel.at[pl.ds(0, CAP)], eids_hbm.at[rrow])
        pltpu.sync_copy(ssl.at[pl.ds(0, CAP)], srcs_hbm.at[rrow])
        pltpu.sync_copy(sdl.at[pl.ds(0, CAP)], dsts_hbm.at[rrow])
        crow[...] = jnp.full((16,), cnt, jnp.int32)
        pltpu.sync_copy(crow, cnts_hbm.at[rrow])

    sort_range(el0, sl0, dl0, lo, o0, 2 * w)
    sort_range(el1, sl1, dl1, mid, o1, 2 * w + 1)


def _sc_prep(src, dst):
    ilist = jax.ShapeDtypeStruct((NRANGE, CAP), jnp.int32)
    k = pl.kernel(
        _prep_body,
        compiler_params=pltpu.CompilerParams(needs_layout_passes=False),
        out_type=[ilist, ilist, ilist,
                  jax.ShapeDtypeStruct((NRANGE, 16), jnp.int32)],
        mesh=_sc_mesh(),
        scratch_types=[
            pltpu.VMEM((ECH,), jnp.int32),
            pltpu.VMEM((ECH,), jnp.int32),
            pltpu.VMEM((CAP16,), jnp.int32),
            pltpu.VMEM((CAP16,), jnp.int32),
            pltpu.VMEM((CAP16,), jnp.int32),
            pltpu.VMEM((CAP16,), jnp.int32),
            pltpu.VMEM((CAP16,), jnp.int32),
            pltpu.VMEM((CAP16,), jnp.int32),
            pltpu.VMEM((CAP16,), jnp.int32),
            pltpu.VMEM((CAP16,), jnp.int32),
            pltpu.VMEM((CAP16,), jnp.int32),
            pltpu.VMEM((HB,), jnp.int32),
            pltpu.VMEM((HB,), jnp.int32),
            pltpu.VMEM((32,), jnp.int32),
            pltpu.VMEM((32,), jnp.int32),
            pltpu.VMEM((16,), jnp.int32),
        ],
    )
    return k(src, dst)


def _stats_body(b_hbm, c_hbm, eids_hbm, srcs_hbm, dsts_hbm, cnts_hbm,
                s_hbm, q_hbm, mx_hbm, mn_hbm, cnt_hbm,
                eidbuf, srcbuf, dstbuf, cbuf, bbuf,
                accS, accQ, accMx, accMn, accC, crow, sem1, sem2):
    w = lax.axis_index("c") * 16 + lax.axis_index("s")

    def do_range(rr, _):
        r = 2 * w + rr
        lo = r * RSZ

        def zero(i, _):
            zf = jnp.zeros((16,), jnp.float32)
            for k in range(8):
                sl = pl.ds(k * 16, 16)
                accS[i, sl] = zf
                accQ[i, sl] = zf
                accMx[i, sl] = jnp.full((16,), NEG, jnp.float32)
                accMn[i, sl] = jnp.full((16,), POS, jnp.float32)
            accC[i, :] = zf
            return 0
        lax.fori_loop(0, RSZ + 1, zero, 0)

        pltpu.sync_copy(cnts_hbm.at[r], crow)
        cnt = jnp.max(crow[...])
        nch = (cnt + (GCH - 1)) // GCH

        zf16 = jnp.zeros((16,), jnp.float32)
        carry0 = (jnp.int32(RSZ), jnp.float32(0.0)) + (zf16,) * 32

        def chunk(g, carry):
            base = g * GCH
            pltpu.sync_copy(eids_hbm.at[r, pl.ds(base, GCH)], eidbuf)
            pltpu.sync_copy(srcs_hbm.at[r, pl.ds(base, GCH)], srcbuf)
            pltpu.sync_copy(dsts_hbm.at[r, pl.ds(base, GCH)],
                            dstbuf.at[pl.ds(0, GCH)])
            cp1 = pltpu.async_copy(c_hbm.at[eidbuf], cbuf, sem1)
            cp2 = pltpu.async_copy(b_hbm.at[srcbuf], bbuf, sem2)
            cp1.wait()
            cp2.wait()
            nin = jnp.minimum(cnt - base, GCH)

            def edge(j, carry):
                prev = carry[0]
                c = carry[1]
                regs = carry[2:]
                d = dstbuf[pl.ds(j, 16)][0] - lo
                is_new = d != prev

                @pl.when(is_new)
                def _():
                    for k in range(8):
                        sl = pl.ds(k * 16, 16)
                        accS[prev, sl] = regs[k]
                        accQ[prev, sl] = regs[8 + k]
                        accMx[prev, sl] = regs[16 + k]
                        accMn[prev, sl] = regs[24 + k]
                    accC[prev, :] = jnp.full((16,), c, jnp.float32)

                ts = [bbuf[j, pl.ds(k * 16, 16)] + cbuf[j, pl.ds(k * 16, 16)]
                      for k in range(8)]
                outS = [jnp.where(is_new, t, s + t)
                        for t, s in zip(ts, regs[0:8])]
                outQ = [jnp.where(is_new, t * t, q + t * t)
                        for t, q in zip(ts, regs[8:16])]
                outX = [jnp.where(is_new, t, jnp.maximum(xx, t))
                        for t, xx in zip(ts, regs[16:24])]
                outN = [jnp.where(is_new, t, jnp.minimum(nn, t))
                        for t, nn in zip(ts, regs[24:32])]
                cn = jnp.where(is_new, jnp.float32(1.0), c + 1.0)
                return (d, cn) + tuple(outS + outQ + outX + outN)

            return lax.fori_loop(0, nin, edge, carry)

        carry = lax.fori_loop(0, nch, chunk, carry0)

        prev = carry[0]
        c = carry[1]
        regs = carry[2:]
        for k in range(8):
            sl = pl.ds(k * 16, 16)
            accS[prev, sl] = regs[k]
            accQ[prev, sl] = regs[8 + k]
            accMx[prev, sl] = regs[16 + k]
            accMn[prev, sl] = regs[24 + k]
        accC[prev, :] = jnp.full((16,), c, jnp.float32)

        pltpu.sync_copy(accS.at[pl.ds(0, RSZ)], s_hbm.at[pl.ds(lo, RSZ)])
        pltpu.sync_copy(accQ.at[pl.ds(0, RSZ)], q_hbm.at[pl.ds(lo, RSZ)])
        pltpu.sync_copy(accMx.at[pl.ds(0, RSZ)], mx_hbm.at[pl.ds(lo, RSZ)])
        pltpu.sync_copy(accMn.at[pl.ds(0, RSZ)], mn_hbm.at[pl.ds(lo, RSZ)])
        pltpu.sync_copy(accC.at[pl.ds(0, RSZ)], cnt_hbm.at[pl.ds(lo, RSZ)])
        return 0

    lax.fori_loop(0, 2, do_range, 0)


def _sc_stats(B, C, eids, srcs, dsts, cnts):
    stat = jax.ShapeDtypeStruct((NP, H), jnp.float32)
    k = pl.kernel(
        _stats_body,
        compiler_params=pltpu.CompilerParams(needs_layout_passes=False),
        out_type=[stat] * 4 + [jax.ShapeDtypeStruct((NP, 16), jnp.float32)],
        mesh=_sc_mesh(),
        scratch_types=[
            pltpu.VMEM((GCH,), jnp.int32),
            pltpu.VMEM((GCH,), jnp.int32),
            pltpu.VMEM((GCH + 16,), jnp.int32),
            pltpu.VMEM((GCH, H), jnp.float32),
            pltpu.VMEM((GCH, H), jnp.float32),
            pltpu.VMEM((RSZ + 1, H), jnp.float32),
            pltpu.VMEM((RSZ + 1, H), jnp.float32),
            pltpu.VMEM((RSZ + 1, H), jnp.float32),
            pltpu.VMEM((RSZ + 1, H), jnp.float32),
            pltpu.VMEM((RSZ + 1, 16), jnp.float32),
            pltpu.VMEM((16,), jnp.int32),
            pltpu.SemaphoreType.DMA,
            pltpu.SemaphoreType.DMA,
        ],
    )
    return k(B, C, eids, srcs, dsts, cnts)


def kernel(x, edge_index, edge_attr, node_W, node_b, edge_W, edge_b,
           pre1_W, pre1_b, post1_W, post1_b, lin1_W, lin1_b, ln1_g, ln1_bt,
           pre2_W, pre2_b, post2_W, post2_b, lin2_W, lin2_b, ln2_g, ln2_bt):
    src = edge_index[0]
    dst = edge_index[1]
    Wd1, Ws1, We1 = pre1_W[:H], pre1_W[H:2 * H], pre1_W[2 * H:]
    Wd2, Ws2, We2 = pre2_W[:H], pre2_W[H:2 * H], pre2_W[2 * H:]

    h, A1, B1 = _node_in(x, node_W, node_b, Wd1, Ws1)
    C1, C2 = _edges(edge_attr, edge_W, edge_b, We1, pre1_b, We2, pre2_b)
    eids, srcs, dsts, cnts = _sc_prep(src, dst)

    S1, Q1, Mx1, Mn1, CNT1 = _sc_stats(B1, C1, eids, srcs, dsts, cnts)
    h1, A2, B2 = _post(h, A1, S1[:N], Q1[:N], Mx1[:N], Mn1[:N], CNT1[:N],
                       post1_W, post1_b, lin1_W, lin1_b, ln1_g, ln1_bt,
                       Wd2, Ws2)

    S2, Q2, Mx2, Mn2, CNT2 = _sc_stats(B2, C2, eids, srcs, dsts, cnts)
    h2, _, _ = _post(h1, A2, S2[:N], Q2[:N], Mx2[:N], Mn2[:N], CNT2[:N],
                     post2_W, post2_b, lin2_W, lin2_b, ln2_g, ln2_bt,
                     Wd2, Ws2)
    return h2


# speedup vs baseline: 1.6118x; 1.0333x over previous
import jax
import jax.numpy as jnp
from jax import lax
from jax.experimental import pallas as pl
from jax.experimental.pallas import tpu as pltpu
from jax.experimental.pallas import tpu_sc as plsc

N = 10000
E = 320000
H = 128
NRANGE = 64
RSZ = 160
NP = NRANGE * RSZ
CAP = 6144
CAP16 = CAP + 16
ECH = 6400
NCH = E // ECH
GCH = 40
HB = 176
NEG = -3.0e38
POS = 3.0e38


def _node_in_body(x_ref, w_ref, b_ref, wd_ref, ws_ref, h_ref, a_ref, b2_ref):
    h = jax.nn.relu(
        jnp.dot(x_ref[...], w_ref[...], preferred_element_type=jnp.float32)
        + b_ref[...])
    h_ref[...] = h
    a_ref[...] = jnp.dot(h, wd_ref[...], preferred_element_type=jnp.float32)
    b2_ref[...] = jnp.dot(h, ws_ref[...], preferred_element_type=jnp.float32)


def _node_in(x, node_W, node_b, Wd, Ws):
    blk = 1000
    full = lambda i: (0, 0)
    return pl.pallas_call(
        _node_in_body,
        grid=(N // blk,),
        in_specs=[
            pl.BlockSpec((blk, H), lambda i: (i, 0)),
            pl.BlockSpec((H, H), full),
            pl.BlockSpec((H,), lambda i: (0,)),
            pl.BlockSpec((H, H), full),
            pl.BlockSpec((H, H), full),
        ],
        out_specs=[pl.BlockSpec((blk, H), lambda i: (i, 0))] * 3,
        out_shape=[jax.ShapeDtypeStruct((N, H), jnp.float32)] * 3,
    )(x, node_W, node_b, Wd, Ws)


def _edges_body(ea_ref, we_ref, be_ref, w1_ref, b1_ref, w2_ref, b2_ref,
                c1_ref, c2_ref):
    e = jax.nn.relu(
        jnp.dot(ea_ref[...], we_ref[...], preferred_element_type=jnp.float32)
        + be_ref[...])
    c1_ref[...] = jnp.dot(e, w1_ref[...], preferred_element_type=jnp.float32) + b1_ref[...]
    c2_ref[...] = jnp.dot(e, w2_ref[...], preferred_element_type=jnp.float32) + b2_ref[...]


def _edges(edge_attr, edge_W, edge_b, We1, pre1_b, We2, pre2_b):
    blk = 2000
    full = lambda i: (0, 0)
    return pl.pallas_call(
        _edges_body,
        grid=(E // blk,),
        in_specs=[
            pl.BlockSpec((blk, 16), lambda i: (i, 0)),
            pl.BlockSpec((16, H), full),
            pl.BlockSpec((H,), lambda i: (0,)),
            pl.BlockSpec((H, H), full),
            pl.BlockSpec((H,), lambda i: (0,)),
            pl.BlockSpec((H, H), full),
            pl.BlockSpec((H,), lambda i: (0,)),
        ],
        out_specs=[pl.BlockSpec((blk, H), lambda i: (i, 0))] * 2,
        out_shape=[jax.ShapeDtypeStruct((E, H), jnp.float32)] * 2,
    )(edge_attr, edge_W, edge_b, We1, pre1_b, We2, pre2_b)


def _post_body(h_ref, a_ref, s_ref, q_ref, mx_ref, mn_ref, cnt_ref,
               pw_ref, pb_ref, lw_ref, lb_ref, g_ref, bt_ref,
               wdn_ref, wsn_ref, hn_ref, an_ref, bn_ref):
    h = h_ref[...]
    A = a_ref[...]
    cnt = cnt_ref[...][:, 0:1]
    invd = 1.0 / jnp.maximum(cnt, 1.0)
    mean_t = s_ref[...] * invd
    var = jax.nn.relu(q_ref[...] * invd - mean_t * mean_t)
    std = jnp.sqrt(var + 1e-5)
    mask = cnt > 0.0
    mean = jnp.where(mask, A + mean_t, 0.0)
    mn = jnp.where(mask, A + mn_ref[...], 0.0)
    mx = jnp.where(mask, A + mx_ref[...], 0.0)
    pw = pw_ref[...]
    dot = lambda a, b: jnp.dot(a, b, preferred_element_type=jnp.float32)
    out = (dot(h, pw[0:H]) + dot(mean, pw[H:2 * H]) + dot(mn, pw[2 * H:3 * H])
           + dot(mx, pw[3 * H:4 * H]) + dot(std, pw[4 * H:5 * H]) + pb_ref[...])
    out = dot(out, lw_ref[...]) + lb_ref[...]
    y = jnp.where(out >= 0.0, out, 0.01 * out)
    mu = jnp.mean(y, axis=-1, keepdims=True)
    v = jnp.mean((y - mu) ** 2, axis=-1, keepdims=True)
    hn = (y - mu) / jnp.sqrt(v + 1e-5) * g_ref[...] + bt_ref[...]
    hn_ref[...] = hn
    an_ref[...] = dot(hn, wdn_ref[...])
    bn_ref[...] = dot(hn, wsn_ref[...])


def _post(h, A, S, Q, Mx, Mn, CNT, post_W, post_b, lin_W, lin_b, g, bt,
          Wdn, Wsn):
    blk = 1000
    full = lambda i: (0, 0)
    vec = lambda i: (0,)
    return pl.pallas_call(
        _post_body,
        grid=(N // blk,),
        in_specs=[
            pl.BlockSpec((blk, H), lambda i: (i, 0)),
            pl.BlockSpec((blk, H), lambda i: (i, 0)),
            pl.BlockSpec((blk, H), lambda i: (i, 0)),
            pl.BlockSpec((blk, H), lambda i: (i, 0)),
            pl.BlockSpec((blk, H), lambda i: (i, 0)),
            pl.BlockSpec((blk, H), lambda i: (i, 0)),
            pl.BlockSpec((blk, 16), lambda i: (i, 0)),
            pl.BlockSpec((5 * H, H), full),
            pl.BlockSpec((H,), vec),
            pl.BlockSpec((H, H), full),
            pl.BlockSpec((H,), vec),
            pl.BlockSpec((H,), vec),
            pl.BlockSpec((H,), vec),
            pl.BlockSpec((H, H), full),
            pl.BlockSpec((H, H), full),
        ],
        out_specs=[pl.BlockSpec((blk, H), lambda i: (i, 0))] * 3,
        out_shape=[jax.ShapeDtypeStruct((N, H), jnp.float32)] * 3,
    )(h, A, S, Q, Mx, Mn, CNT, post_W, post_b, lin_W, lin_b, g, bt, Wdn, Wsn)


def _sc_mesh():
    return plsc.VectorSubcoreMesh(core_axis_name="c", subcore_axis_name="s")


def _prep_body(src_hbm, dst_hbm, eids_hbm, srcs_hbm, dsts_hbm, cnts_hbm,
               dstbuf, srcbuf, el0, sl0, dl0, el1, sl1, dl1,
               sel, ssl, sdl, hist, off, tmpa, tmpb, crow):
    w = lax.axis_index("c") * 16 + lax.axis_index("s")
    lo = w * (2 * RSZ)
    mid = lo + RSZ
    hi = lo + 2 * RSZ
    IOTA = lax.iota(jnp.int32, 16)
    ZER = jnp.zeros((16,), jnp.int32)
    ONE = jnp.full((16,), 1, jnp.int32)

    def zero(i, _):
        z = jnp.zeros((16,), jnp.int32)
        sl = pl.ds(i * 16, 16)
        el0[sl] = z
        sl0[sl] = z
        dl0[sl] = z
        el1[sl] = z
        sl1[sl] = z
        dl1[sl] = z
        sel[sl] = z
        ssl[sl] = z
        sdl[sl] = z
        return 0
    lax.fori_loop(0, CAP16 // 16, zero, 0)

    def chunk(ci, offs):
        pltpu.sync_copy(dst_hbm.at[pl.ds(ci * ECH, ECH)], dstbuf)
        pltpu.sync_copy(src_hbm.at[pl.ds(ci * ECH, ECH)], srcbuf)

        def step(si, offs):
            o0, o1 = offs
            sl = pl.ds(si * 16, 16)
            d16 = dstbuf[sl]
            s16 = srcbuf[sl]
            e16 = ci * ECH + si * 16 + IOTA
            m0 = (d16 >= lo) & (d16 < mid)
            m1 = (d16 >= mid) & (d16 < hi)
            n0 = plsc.all_reduce_population_count(m0)[0]
            n1 = plsc.all_reduce_population_count(m1)[0]

            @pl.when(n0 > 0)
            def _():
                plsc.store_compressed(el0.at[pl.ds(o0, 16)], e16, mask=m0)
                plsc.store_compressed(sl0.at[pl.ds(o0, 16)], s16, mask=m0)
                plsc.store_compressed(dl0.at[pl.ds(o0, 16)], d16, mask=m0)

            @pl.when(n1 > 0)
            def _():
                plsc.store_compressed(el1.at[pl.ds(o1, 16)], e16, mask=m1)
                plsc.store_compressed(sl1.at[pl.ds(o1, 16)], s16, mask=m1)
                plsc.store_compressed(dl1.at[pl.ds(o1, 16)], d16, mask=m1)

            return (jnp.minimum(o0 + n0, CAP), jnp.minimum(o1 + n1, CAP))

        return lax.fori_loop(0, ECH // 16, step, offs)

    o0, o1 = lax.fori_loop(0, NCH, chunk, (0, 0))

    tmpb[pl.ds(16, 16)] = ONE

    def sort_range(elX, slX, dlX, rlo, cnt, rrow):
        def zh(i, _):
            hist[pl.ds(i * 16, 16)] = ZER
            return 0
        lax.fori_loop(0, HB // 16, zh, 0)

        ngroups = (cnt + 15) // 16

        def keys_of(gi):
            jb = gi * 16
            draw = dlX[pl.ds(jb, 16)] - rlo
            valid = (jb + IOTA) < cnt
            d16 = jnp.where(valid, draw, jnp.full((16,), RSZ, jnp.int32))
            sd, perm = plsc.sort_key_val(d16, IOTA)
            tmpa[pl.ds(0, 16)] = jnp.full((16,), -1, jnp.int32)
            tmpa[pl.ds(1, 16)] = sd
            prev = tmpa[pl.ds(0, 16)]
            is_start = sd != prev
            runstart = plsc.cummax(jnp.where(is_start, IOTA, ZER))
            rank = IOTA - runstart
            tmpb[pl.ds(0, 16)] = jnp.where(is_start, ONE, ZER)
            is_last = tmpb[pl.ds(1, 16)] > 0
            return jb, sd, perm, rank, is_last

        def hgroup(gi, _):
            _, sd, _, rank, is_last = keys_of(gi)
            old = plsc.load_gather(hist, [sd])
            plsc.store_scatter(hist, [sd], old + rank + 1, mask=is_last)
            return 0
        lax.fori_loop(0, ngroups, hgroup, 0)

        carry = 0
        for gi in range(HB // 16):
            v = hist[pl.ds(gi * 16, 16)]
            incl = plsc.cumsum(v)
            off[pl.ds(gi * 16, 16)] = incl - v + carry
            carry = carry + incl[15]

        def pgroup(gi, _):
            jb, sd, perm, rank, is_last = keys_of(gi)
            base = plsc.load_gather(off, [sd])
            pos = base + rank
            gidx = perm + jb
            plsc.store_scatter(sel, [pos], plsc.load_gather(elX, [gidx]))
            plsc.store_scatter(ssl, [pos], plsc.load_gather(slX, [gidx]))
            plsc.store_scatter(sdl, [pos], plsc.load_gather(dlX, [gidx]))
            plsc.store_scatter(off, [sd], pos + 1, mask=is_last)
            return 0
        lax.fori_loop(0, ngroups, pgroup, 0)

        pltpu.sync_copy(sel.at[pl.ds(0, CAP)],
                        eids_hbm.at[pl.ds(rrow * CAP, CAP)])
        pltpu.sync_copy(ssl.at[pl.ds(0, CAP)],
                        srcs_hbm.at[pl.ds(rrow * CAP, CAP)])
        pltpu.sync_copy(sdl.at[pl.ds(0, CAP)],
                        dsts_hbm.at[pl.ds(rrow * CAP, CAP)])
        crow[...] = jnp.full((16,), cnt, jnp.int32)
        pltpu.sync_copy(crow, cnts_hbm.at[pl.ds(rrow * 16, 16)])

    sort_range(el0, sl0, dl0, lo, o0, 2 * w)
    sort_range(el1, sl1, dl1, mid, o1, 2 * w + 1)


def _sc_prep(src, dst):
    ilist = jax.ShapeDtypeStruct((NRANGE * CAP,), jnp.int32)
    k = pl.kernel(
        _prep_body,
        compiler_params=pltpu.CompilerParams(needs_layout_passes=False),
        out_type=[ilist, ilist, ilist,
                  jax.ShapeDtypeStruct((NRANGE * 16,), jnp.int32)],
        mesh=_sc_mesh(),
        scratch_types=[
            pltpu.VMEM((ECH,), jnp.int32),
            pltpu.VMEM((ECH,), jnp.int32),
            pltpu.VMEM((CAP16,), jnp.int32),
            pltpu.VMEM((CAP16,), jnp.int32),
            pltpu.VMEM((CAP16,), jnp.int32),
            pltpu.VMEM((CAP16,), jnp.int32),
            pltpu.VMEM((CAP16,), jnp.int32),
            pltpu.VMEM((CAP16,), jnp.int32),
            pltpu.VMEM((CAP16,), jnp.int32),
            pltpu.VMEM((CAP16,), jnp.int32),
            pltpu.VMEM((CAP16,), jnp.int32),
            pltpu.VMEM((HB,), jnp.int32),
            pltpu.VMEM((HB,), jnp.int32),
            pltpu.VMEM((32,), jnp.int32),
            pltpu.VMEM((32,), jnp.int32),
            pltpu.VMEM((16,), jnp.int32),
        ],
    )
    return k(src, dst)


def _stats_body(b_hbm, c_hbm, eids_hbm, srcs_hbm, dsts_hbm, cnts_hbm,
                s_hbm, q_hbm, mx_hbm, mn_hbm, cnt_hbm,
                eidbufA, srcbufA, dstbufA, cbufA, bbufA,
                eidbufB, srcbufB, dstbufB, cbufB, bbufB,
                accS, accQ, accMx, accMn, accC, crow,
                semA1, semA2, semB1, semB2):
    w = lax.axis_index("c") * 16 + lax.axis_index("s")
    bufA = (eidbufA, srcbufA, dstbufA, cbufA, bbufA, semA1, semA2)
    bufB = (eidbufB, srcbufB, dstbufB, cbufB, bbufB, semB1, semB2)

    def do_range(rr, _):
        r = 2 * w + rr
        lo = r * RSZ

        def zero(i, _):
            zf = jnp.zeros((16,), jnp.float32)
            for k in range(8):
                sl = pl.ds(k * 16, 16)
                accS[i, sl] = zf
                accQ[i, sl] = zf
                accMx[i, sl] = jnp.full((16,), NEG, jnp.float32)
                accMn[i, sl] = jnp.full((16,), POS, jnp.float32)
            accC[i, :] = zf
            return 0
        lax.fori_loop(0, RSZ + 1, zero, 0)

        pltpu.sync_copy(cnts_hbm.at[pl.ds(pl.multiple_of(r * 16, 8), 16)],
                        crow)
        cnt = jnp.max(crow[...])
        nch = (cnt + (GCH - 1)) // GCH

        zf16 = jnp.zeros((16,), jnp.float32)
        carry0 = (jnp.int32(RSZ), jnp.float32(0.0)) + (zf16,) * 32

        def stage(g, bufs):
            eidbuf, srcbuf, dstbuf, cbuf, bbuf, s1, s2 = bufs
            base = pl.multiple_of(
                r * CAP + jnp.maximum(0, jnp.minimum(g, nch - 1)) * GCH, 8)
            pltpu.sync_copy(eids_hbm.at[pl.ds(base, GCH)], eidbuf)
            pltpu.sync_copy(srcs_hbm.at[pl.ds(base, GCH)], srcbuf)
            pltpu.sync_copy(dsts_hbm.at[pl.ds(base, GCH)],
                            dstbuf.at[pl.ds(0, GCH)])
            pltpu.async_copy(c_hbm.at[eidbuf], cbuf, s1)
            pltpu.async_copy(b_hbm.at[srcbuf], bbuf, s2)

        def drain(bufs):
            eidbuf, srcbuf, dstbuf, cbuf, bbuf, s1, s2 = bufs
            pltpu.make_async_copy(c_hbm.at[eidbuf], cbuf, s1).wait()
            pltpu.make_async_copy(b_hbm.at[srcbuf], bbuf, s2).wait()

        def process(g, bufs, carry):
            eidbuf, srcbuf, dstbuf, cbuf, bbuf, s1, s2 = bufs
            nin = jnp.maximum(0, jnp.minimum(cnt - g * GCH, GCH))

            def edge(j, carry):
                prev = carry[0]
                c = carry[1]
                regs = carry[2:]
                d = dstbuf[pl.ds(j, 16)][0] - lo
                is_new = d != prev

                @pl.when(is_new)
                def _():
                    for k in range(8):
                        sl = pl.ds(k * 16, 16)
                        accS[prev, sl] = regs[k]
                        accQ[prev, sl] = regs[8 + k]
                        accMx[prev, sl] = regs[16 + k]
                        accMn[prev, sl] = regs[24 + k]
                    accC[prev, :] = jnp.full((16,), c, jnp.float32)

                ts = [bbuf[j, pl.ds(k * 16, 16)] + cbuf[j, pl.ds(k * 16, 16)]
                      for k in range(8)]
                outS = [jnp.where(is_new, t, s + t)
                        for t, s in zip(ts, regs[0:8])]
                outQ = [jnp.where(is_new, t * t, q + t * t)
                        for t, q in zip(ts, regs[8:16])]
                outX = [jnp.where(is_new, t, jnp.maximum(xx, t))
                        for t, xx in zip(ts, regs[16:24])]
                outN = [jnp.where(is_new, t, jnp.minimum(nn, t))
                        for t, nn in zip(ts, regs[24:32])]
                cn = jnp.where(is_new, jnp.float32(1.0), c + 1.0)
                return (d, cn) + tuple(outS + outQ + outX + outN)

            return lax.fori_loop(0, nin, edge, carry)

        stage(0, bufA)
        npair = (nch + 1) // 2

        def pair(p, carry):
            g0 = 2 * p
            stage(g0 + 1, bufB)
            drain(bufA)
            carry = process(g0, bufA, carry)
            stage(g0 + 2, bufA)
            drain(bufB)
            carry = process(g0 + 1, bufB, carry)
            return carry

        carry = lax.fori_loop(0, npair, pair, carry0)
        drain(bufA)

        prev = carry[0]
        c = carry[1]
        regs = carry[2:]
        for k in range(8):
            sl = pl.ds(k * 16, 16)
            accS[prev, sl] = regs[k]
            accQ[prev, sl] = regs[8 + k]
            accMx[prev, sl] = regs[16 + k]
            accMn[prev, sl] = regs[24 + k]
        accC[prev, :] = jnp.full((16,), c, jnp.float32)

        pltpu.sync_copy(accS.at[pl.ds(0, RSZ)], s_hbm.at[pl.ds(lo, RSZ)])
        pltpu.sync_copy(accQ.at[pl.ds(0, RSZ)], q_hbm.at[pl.ds(lo, RSZ)])
        pltpu.sync_copy(accMx.at[pl.ds(0, RSZ)], mx_hbm.at[pl.ds(lo, RSZ)])
        pltpu.sync_copy(accMn.at[pl.ds(0, RSZ)], mn_hbm.at[pl.ds(lo, RSZ)])
        pltpu.sync_copy(accC.at[pl.ds(0, RSZ)], cnt_hbm.at[pl.ds(lo, RSZ)])
        return 0

    lax.fori_loop(0, 2, do_range, 0)


def _sc_stats(B, C, eids, srcs, dsts, cnts):
    stat = jax.ShapeDtypeStruct((NP, H), jnp.float32)
    k = pl.kernel(
        _stats_body,
        compiler_params=pltpu.CompilerParams(needs_layout_passes=False),
        out_type=[stat] * 4 + [jax.ShapeDtypeStruct((NP, 16), jnp.float32)],
        mesh=_sc_mesh(),
        scratch_types=[
            pltpu.VMEM((GCH,), jnp.int32),
            pltpu.VMEM((GCH,), jnp.int32),
            pltpu.VMEM((GCH + 16,), jnp.int32),
            pltpu.VMEM((GCH, H), jnp.float32),
            pltpu.VMEM((GCH, H), jnp.float32),
            pltpu.VMEM((GCH,), jnp.int32),
            pltpu.VMEM((GCH,), jnp.int32),
            pltpu.VMEM((GCH + 16,), jnp.int32),
            pltpu.VMEM((GCH, H), jnp.float32),
            pltpu.VMEM((GCH, H), jnp.float32),
            pltpu.VMEM((RSZ + 1, H), jnp.float32),
            pltpu.VMEM((RSZ + 1, H), jnp.float32),
            pltpu.VMEM((RSZ + 1, H), jnp.float32),
            pltpu.VMEM((RSZ + 1, H), jnp.float32),
            pltpu.VMEM((RSZ + 1, 16), jnp.float32),
            pltpu.VMEM((16,), jnp.int32),
            pltpu.SemaphoreType.DMA,
            pltpu.SemaphoreType.DMA,
            pltpu.SemaphoreType.DMA,
            pltpu.SemaphoreType.DMA,
        ],
    )
    return k(B, C, eids, srcs, dsts, cnts)


def kernel(x, edge_index, edge_attr, node_W, node_b, edge_W, edge_b,
           pre1_W, pre1_b, post1_W, post1_b, lin1_W, lin1_b, ln1_g, ln1_bt,
           pre2_W, pre2_b, post2_W, post2_b, lin2_W, lin2_b, ln2_g, ln2_bt):
    src = edge_index[0]
    dst = edge_index[1]
    Wd1, Ws1, We1 = pre1_W[:H], pre1_W[H:2 * H], pre1_W[2 * H:]
    Wd2, Ws2, We2 = pre2_W[:H], pre2_W[H:2 * H], pre2_W[2 * H:]

    h, A1, B1 = _node_in(x, node_W, node_b, Wd1, Ws1)
    C1, C2 = _edges(edge_attr, edge_W, edge_b, We1, pre1_b, We2, pre2_b)
    eids, srcs, dsts, cnts = _sc_prep(src, dst)

    S1, Q1, Mx1, Mn1, CNT1 = _sc_stats(B1, C1, eids, srcs, dsts, cnts)
    h1, A2, B2 = _post(h, A1, S1[:N], Q1[:N], Mx1[:N], Mn1[:N], CNT1[:N],
                       post1_W, post1_b, lin1_W, lin1_b, ln1_g, ln1_bt,
                       Wd2, Ws2)

    S2, Q2, Mx2, Mn2, CNT2 = _sc_stats(B2, C2, eids, srcs, dsts, cnts)
    h2, _, _ = _post(h1, A2, S2[:N], Q2[:N], Mx2[:N], Mn2[:N], CNT2[:N],
                     post2_W, post2_b, lin2_W, lin2_b, ln2_g, ln2_bt,
                     Wd2, Ws2)
    return h2


# speedup vs baseline: 1.7414x; 1.0804x over previous
import jax
import jax.numpy as jnp
from jax import lax
from jax.experimental import pallas as pl
from jax.experimental.pallas import tpu as pltpu
from jax.experimental.pallas import tpu_sc as plsc

N = 10000
E = 320000
H = 128
NRANGE = 64
RSZ = 160
NP = NRANGE * RSZ
CAP = 6144
CAP16 = CAP + 16
ECH = 6400
NCH = E // ECH
GCH = 40
HB = 176
NEG = -3.0e38
POS = 3.0e38


def _node_in_body(x_ref, w_ref, b_ref, wd_ref, ws_ref, h_ref, a_ref, b2_ref):
    h = jax.nn.relu(
        jnp.dot(x_ref[...], w_ref[...], preferred_element_type=jnp.float32)
        + b_ref[...])
    h_ref[...] = h
    a_ref[...] = jnp.dot(h, wd_ref[...], preferred_element_type=jnp.float32)
    b2_ref[...] = jnp.dot(h, ws_ref[...], preferred_element_type=jnp.float32)


def _node_in(x, node_W, node_b, Wd, Ws):
    blk = 1000
    full = lambda i: (0, 0)
    return pl.pallas_call(
        _node_in_body,
        grid=(N // blk,),
        in_specs=[
            pl.BlockSpec((blk, H), lambda i: (i, 0)),
            pl.BlockSpec((H, H), full),
            pl.BlockSpec((H,), lambda i: (0,)),
            pl.BlockSpec((H, H), full),
            pl.BlockSpec((H, H), full),
        ],
        out_specs=[pl.BlockSpec((blk, H), lambda i: (i, 0))] * 3,
        out_shape=[jax.ShapeDtypeStruct((N, H), jnp.float32)] * 3,
    )(x, node_W, node_b, Wd, Ws)


def _edges_body(ea_ref, we_ref, be_ref, w1_ref, b1_ref, w2_ref, b2_ref,
                c1_ref, c2_ref):
    e = jax.nn.relu(
        jnp.dot(ea_ref[...], we_ref[...], preferred_element_type=jnp.float32)
        + be_ref[...])
    c1_ref[...] = jnp.dot(e, w1_ref[...], preferred_element_type=jnp.float32) + b1_ref[...]
    c2_ref[...] = jnp.dot(e, w2_ref[...], preferred_element_type=jnp.float32) + b2_ref[...]


def _edges(edge_attr, edge_W, edge_b, We1, pre1_b, We2, pre2_b):
    blk = 2000
    full = lambda i: (0, 0)
    return pl.pallas_call(
        _edges_body,
        grid=(E // blk,),
        in_specs=[
            pl.BlockSpec((blk, 16), lambda i: (i, 0)),
            pl.BlockSpec((16, H), full),
            pl.BlockSpec((H,), lambda i: (0,)),
            pl.BlockSpec((H, H), full),
            pl.BlockSpec((H,), lambda i: (0,)),
            pl.BlockSpec((H, H), full),
            pl.BlockSpec((H,), lambda i: (0,)),
        ],
        out_specs=[pl.BlockSpec((blk, H), lambda i: (i, 0))] * 2,
        out_shape=[jax.ShapeDtypeStruct((E, H), jnp.float32)] * 2,
    )(edge_attr, edge_W, edge_b, We1, pre1_b, We2, pre2_b)


def _post_body(h_ref, a_ref, s_ref, q_ref, mx_ref, mn_ref, cnt_ref,
               pw_ref, pb_ref, lw_ref, lb_ref, g_ref, bt_ref,
               wdn_ref, wsn_ref, hn_ref, an_ref, bn_ref):
    h = h_ref[...]
    A = a_ref[...]
    cnt = cnt_ref[...][:, 0:1]
    invd = 1.0 / jnp.maximum(cnt, 1.0)
    mean_t = s_ref[...] * invd
    var = jax.nn.relu(q_ref[...] * invd - mean_t * mean_t)
    std = jnp.sqrt(var + 1e-5)
    mask = cnt > 0.0
    mean = jnp.where(mask, A + mean_t, 0.0)
    mn = jnp.where(mask, A + mn_ref[...], 0.0)
    mx = jnp.where(mask, A + mx_ref[...], 0.0)
    pw = pw_ref[...]
    dot = lambda a, b: jnp.dot(a, b, preferred_element_type=jnp.float32)
    out = (dot(h, pw[0:H]) + dot(mean, pw[H:2 * H]) + dot(mn, pw[2 * H:3 * H])
           + dot(mx, pw[3 * H:4 * H]) + dot(std, pw[4 * H:5 * H]) + pb_ref[...])
    out = dot(out, lw_ref[...]) + lb_ref[...]
    y = jnp.where(out >= 0.0, out, 0.01 * out)
    mu = jnp.mean(y, axis=-1, keepdims=True)
    v = jnp.mean((y - mu) ** 2, axis=-1, keepdims=True)
    hn = (y - mu) / jnp.sqrt(v + 1e-5) * g_ref[...] + bt_ref[...]
    hn_ref[...] = hn
    an_ref[...] = dot(hn, wdn_ref[...])
    bn_ref[...] = dot(hn, wsn_ref[...])


def _post(h, A, S, Q, Mx, Mn, CNT, post_W, post_b, lin_W, lin_b, g, bt,
          Wdn, Wsn):
    blk = 1000
    full = lambda i: (0, 0)
    vec = lambda i: (0,)
    return pl.pallas_call(
        _post_body,
        grid=(N // blk,),
        in_specs=[
            pl.BlockSpec((blk, H), lambda i: (i, 0)),
            pl.BlockSpec((blk, H), lambda i: (i, 0)),
            pl.BlockSpec((blk, H), lambda i: (i, 0)),
            pl.BlockSpec((blk, H), lambda i: (i, 0)),
            pl.BlockSpec((blk, H), lambda i: (i, 0)),
            pl.BlockSpec((blk, H), lambda i: (i, 0)),
            pl.BlockSpec((blk, 16), lambda i: (i, 0)),
            pl.BlockSpec((5 * H, H), full),
            pl.BlockSpec((H,), vec),
            pl.BlockSpec((H, H), full),
            pl.BlockSpec((H,), vec),
            pl.BlockSpec((H,), vec),
            pl.BlockSpec((H,), vec),
            pl.BlockSpec((H, H), full),
            pl.BlockSpec((H, H), full),
        ],
        out_specs=[pl.BlockSpec((blk, H), lambda i: (i, 0))] * 3,
        out_shape=[jax.ShapeDtypeStruct((N, H), jnp.float32)] * 3,
    )(h, A, S, Q, Mx, Mn, CNT, post_W, post_b, lin_W, lin_b, g, bt, Wdn, Wsn)


def _sc_mesh():
    return plsc.VectorSubcoreMesh(core_axis_name="c", subcore_axis_name="s")


def _prep_body(src_hbm, dst_hbm, eids_hbm, srcs_hbm, dsts_hbm, cnts_hbm,
               dstbuf, srcbuf, el0, sl0, dl0, el1, sl1, dl1,
               sel, ssl, sdl, hist, off, tmpa, tmpb, crow):
    w = lax.axis_index("c") * 16 + lax.axis_index("s")
    lo = w * (2 * RSZ)
    mid = lo + RSZ
    hi = lo + 2 * RSZ
    IOTA = lax.iota(jnp.int32, 16)
    ZER = jnp.zeros((16,), jnp.int32)
    ONE = jnp.full((16,), 1, jnp.int32)

    def zero(i, _):
        z = jnp.zeros((16,), jnp.int32)
        sl = pl.ds(i * 16, 16)
        el0[sl] = z
        sl0[sl] = z
        dl0[sl] = z
        el1[sl] = z
        sl1[sl] = z
        dl1[sl] = z
        sel[sl] = z
        ssl[sl] = z
        sdl[sl] = z
        return 0
    lax.fori_loop(0, CAP16 // 16, zero, 0)

    def chunk(ci, offs):
        pltpu.sync_copy(dst_hbm.at[pl.ds(ci * ECH, ECH)], dstbuf)
        pltpu.sync_copy(src_hbm.at[pl.ds(ci * ECH, ECH)], srcbuf)

        def step(si, offs):
            o0, o1 = offs
            sl = pl.ds(si * 16, 16)
            d16 = dstbuf[sl]
            s16 = srcbuf[sl]
            e16 = ci * ECH + si * 16 + IOTA
            m0 = (d16 >= lo) & (d16 < mid)
            m1 = (d16 >= mid) & (d16 < hi)
            n0 = plsc.all_reduce_population_count(m0)[0]
            n1 = plsc.all_reduce_population_count(m1)[0]

            @pl.when(n0 > 0)
            def _():
                plsc.store_compressed(el0.at[pl.ds(o0, 16)], e16, mask=m0)
                plsc.store_compressed(sl0.at[pl.ds(o0, 16)], s16, mask=m0)
                plsc.store_compressed(dl0.at[pl.ds(o0, 16)], d16, mask=m0)

            @pl.when(n1 > 0)
            def _():
                plsc.store_compressed(el1.at[pl.ds(o1, 16)], e16, mask=m1)
                plsc.store_compressed(sl1.at[pl.ds(o1, 16)], s16, mask=m1)
                plsc.store_compressed(dl1.at[pl.ds(o1, 16)], d16, mask=m1)

            return (jnp.minimum(o0 + n0, CAP), jnp.minimum(o1 + n1, CAP))

        return lax.fori_loop(0, ECH // 16, step, offs)

    o0, o1 = lax.fori_loop(0, NCH, chunk, (0, 0))

    tmpb[pl.ds(16, 16)] = ONE

    def sort_range(elX, slX, dlX, rlo, cnt, rrow):
        def zh(i, _):
            hist[pl.ds(i * 16, 16)] = ZER
            return 0
        lax.fori_loop(0, HB // 16, zh, 0)

        ngroups = (cnt + 15) // 16

        def keys_of(gi):
            jb = gi * 16
            draw = dlX[pl.ds(jb, 16)] - rlo
            valid = (jb + IOTA) < cnt
            d16 = jnp.where(valid, draw, jnp.full((16,), RSZ, jnp.int32))
            sd, perm = plsc.sort_key_val(d16, IOTA)
            tmpa[pl.ds(0, 16)] = jnp.full((16,), -1, jnp.int32)
            tmpa[pl.ds(1, 16)] = sd
            prev = tmpa[pl.ds(0, 16)]
            is_start = sd != prev
            runstart = plsc.cummax(jnp.where(is_start, IOTA, ZER))
            rank = IOTA - runstart
            tmpb[pl.ds(0, 16)] = jnp.where(is_start, ONE, ZER)
            is_last = tmpb[pl.ds(1, 16)] > 0
            return jb, sd, perm, rank, is_last

        def hgroup(gi, _):
            _, sd, _, rank, is_last = keys_of(gi)
            old = plsc.load_gather(hist, [sd])
            plsc.store_scatter(hist, [sd], old + rank + 1, mask=is_last)
            return 0
        lax.fori_loop(0, ngroups, hgroup, 0)

        carry = 0
        for gi in range(HB // 16):
            v = hist[pl.ds(gi * 16, 16)]
            incl = plsc.cumsum(v)
            off[pl.ds(gi * 16, 16)] = incl - v + carry
            carry = carry + incl[15]

        def pgroup(gi, _):
            jb, sd, perm, rank, is_last = keys_of(gi)
            base = plsc.load_gather(off, [sd])
            pos = base + rank
            gidx = perm + jb
            plsc.store_scatter(sel, [pos], plsc.load_gather(elX, [gidx]))
            plsc.store_scatter(ssl, [pos], plsc.load_gather(slX, [gidx]))
            plsc.store_scatter(sdl, [pos], plsc.load_gather(dlX, [gidx]))
            plsc.store_scatter(off, [sd], pos + 1, mask=is_last)
            return 0
        lax.fori_loop(0, ngroups, pgroup, 0)

        pltpu.sync_copy(sel.at[pl.ds(0, CAP)],
                        eids_hbm.at[pl.ds(rrow * CAP, CAP)])
        pltpu.sync_copy(ssl.at[pl.ds(0, CAP)],
                        srcs_hbm.at[pl.ds(rrow * CAP, CAP)])
        pltpu.sync_copy(sdl.at[pl.ds(0, CAP)],
                        dsts_hbm.at[pl.ds(rrow * CAP, CAP)])
        crow[...] = jnp.full((16,), cnt, jnp.int32)
        pltpu.sync_copy(crow, cnts_hbm.at[pl.ds(rrow * 16, 16)])

    sort_range(el0, sl0, dl0, lo, o0, 2 * w)
    sort_range(el1, sl1, dl1, mid, o1, 2 * w + 1)


def _sc_prep(src, dst):
    ilist = jax.ShapeDtypeStruct((NRANGE * CAP,), jnp.int32)
    k = pl.kernel(
        _prep_body,
        compiler_params=pltpu.CompilerParams(needs_layout_passes=False),
        out_type=[ilist, ilist, ilist,
                  jax.ShapeDtypeStruct((NRANGE * 16,), jnp.int32)],
        mesh=_sc_mesh(),
        scratch_types=[
            pltpu.VMEM((ECH,), jnp.int32),
            pltpu.VMEM((ECH,), jnp.int32),
            pltpu.VMEM((CAP16,), jnp.int32),
            pltpu.VMEM((CAP16,), jnp.int32),
            pltpu.VMEM((CAP16,), jnp.int32),
            pltpu.VMEM((CAP16,), jnp.int32),
            pltpu.VMEM((CAP16,), jnp.int32),
            pltpu.VMEM((CAP16,), jnp.int32),
            pltpu.VMEM((CAP16,), jnp.int32),
            pltpu.VMEM((CAP16,), jnp.int32),
            pltpu.VMEM((CAP16,), jnp.int32),
            pltpu.VMEM((HB,), jnp.int32),
            pltpu.VMEM((HB,), jnp.int32),
            pltpu.VMEM((32,), jnp.int32),
            pltpu.VMEM((32,), jnp.int32),
            pltpu.VMEM((16,), jnp.int32),
        ],
    )
    return k(src, dst)


def _stats_body(b_hbm, c_hbm, eids_hbm, srcs_hbm, dsts_hbm, cnts_hbm,
                s_hbm, q_hbm, mx_hbm, mn_hbm, cnt_hbm,
                eidbufA, srcbufA, dstbufA, cbufA, bbufA,
                eidbufB, srcbufB, dstbufB, cbufB, bbufB,
                accS, accQ, accMx, accMn, accC, crow,
                semA1, semA2, semB1, semB2):
    w = lax.axis_index("c") * 16 + lax.axis_index("s")
    bufA = (eidbufA, srcbufA, dstbufA, cbufA, bbufA, semA1, semA2)
    bufB = (eidbufB, srcbufB, dstbufB, cbufB, bbufB, semB1, semB2)

    def do_range(rr, _):
        r = 2 * w + rr
        lo = r * RSZ

        def zero(i, _):
            zf = jnp.zeros((16,), jnp.float32)
            for k in range(8):
                sl = pl.ds(k * 16, 16)
                accS[i, sl] = zf
                accQ[i, sl] = zf
                accMx[i, sl] = jnp.full((16,), NEG, jnp.float32)
                accMn[i, sl] = jnp.full((16,), POS, jnp.float32)
            accC[i, :] = zf
            return 0
        lax.fori_loop(0, RSZ + 1, zero, 0)

        pltpu.sync_copy(cnts_hbm.at[pl.ds(pl.multiple_of(r * 16, 8), 16)],
                        crow)
        cnt = jnp.max(crow[...])
        nch = (cnt + (GCH - 1)) // GCH

        zf16 = jnp.zeros((16,), jnp.float32)
        carry0 = (jnp.int32(RSZ), jnp.float32(0.0)) + (zf16,) * 32

        def stage(g, bufs):
            eidbuf, srcbuf, dstbuf, cbuf, bbuf, s1, s2 = bufs
            base = pl.multiple_of(
                r * CAP + jnp.maximum(0, jnp.minimum(g, nch - 1)) * GCH, 8)
            pltpu.sync_copy(eids_hbm.at[pl.ds(base, GCH)], eidbuf)
            pltpu.sync_copy(srcs_hbm.at[pl.ds(base, GCH)], srcbuf)
            pltpu.sync_copy(dsts_hbm.at[pl.ds(base, GCH)],
                            dstbuf.at[pl.ds(0, GCH)])
            pltpu.async_copy(c_hbm.at[eidbuf], cbuf, s1)
            pltpu.async_copy(b_hbm.at[srcbuf], bbuf, s2)

        def drain(bufs):
            eidbuf, srcbuf, dstbuf, cbuf, bbuf, s1, s2 = bufs
            pltpu.make_async_copy(c_hbm.at[eidbuf], cbuf, s1).wait()
            pltpu.make_async_copy(b_hbm.at[srcbuf], bbuf, s2).wait()

        def process(g, bufs, carry):
            eidbuf, srcbuf, dstbuf, cbuf, bbuf, s1, s2 = bufs
            nin = jnp.maximum(0, jnp.minimum(cnt - g * GCH, GCH))

            def half_edge(half, with_cnt):
                ko = 4 * half

                def edge(j, hc):
                    prev = hc[0]
                    c = hc[1]
                    regs = hc[2:]
                    d = dstbuf[pl.ds(j, 16)][0] - lo
                    is_new = d != prev

                    @pl.when(is_new)
                    def _():
                        for k in range(4):
                            sl = pl.ds((ko + k) * 16, 16)
                            accS[prev, sl] = regs[k]
                            accQ[prev, sl] = regs[4 + k]
                            accMx[prev, sl] = regs[8 + k]
                            accMn[prev, sl] = regs[12 + k]
                        if with_cnt:
                            accC[prev, :] = jnp.full((16,), c, jnp.float32)

                    ts = [bbuf[j, pl.ds((ko + k) * 16, 16)]
                          + cbuf[j, pl.ds((ko + k) * 16, 16)]
                          for k in range(4)]
                    outS = [jnp.where(is_new, t, s + t)
                            for t, s in zip(ts, regs[0:4])]
                    outQ = [jnp.where(is_new, t * t, q + t * t)
                            for t, q in zip(ts, regs[4:8])]
                    outX = [jnp.where(is_new, t, jnp.maximum(xx, t))
                            for t, xx in zip(ts, regs[8:12])]
                    outN = [jnp.where(is_new, t, jnp.minimum(nn, t))
                            for t, nn in zip(ts, regs[12:16])]
                    cn = jnp.where(is_new, jnp.float32(1.0), c + 1.0)
                    return (d, cn) + tuple(outS + outQ + outX + outN)
                return edge

            h0 = (carry[0], carry[1]) + carry[2:6] + carry[10:14] \
                + carry[18:22] + carry[26:30]
            h1 = (carry[0], carry[1]) + carry[6:10] + carry[14:18] \
                + carry[22:26] + carry[30:34]
            h0 = lax.fori_loop(0, nin, half_edge(0, True), h0)
            h1 = lax.fori_loop(0, nin, half_edge(1, False), h1)
            return (h0[0], h0[1]) + h0[2:6] + h1[2:6] + h0[6:10] \
                + h1[6:10] + h0[10:14] + h1[10:14] + h0[14:18] + h1[14:18]

        stage(0, bufA)
        npair = (nch + 1) // 2

        def pair(p, carry):
            g0 = 2 * p
            stage(g0 + 1, bufB)
            drain(bufA)
            carry = process(g0, bufA, carry)
            stage(g0 + 2, bufA)
            drain(bufB)
            carry = process(g0 + 1, bufB, carry)
            return carry

        carry = lax.fori_loop(0, npair, pair, carry0)
        drain(bufA)

        prev = carry[0]
        c = carry[1]
        regs = carry[2:]
        for k in range(8):
            sl = pl.ds(k * 16, 16)
            accS[prev, sl] = regs[k]
            accQ[prev, sl] = regs[8 + k]
            accMx[prev, sl] = regs[16 + k]
            accMn[prev, sl] = regs[24 + k]
        accC[prev, :] = jnp.full((16,), c, jnp.float32)

        pltpu.sync_copy(accS.at[pl.ds(0, RSZ)], s_hbm.at[pl.ds(lo, RSZ)])
        pltpu.sync_copy(accQ.at[pl.ds(0, RSZ)], q_hbm.at[pl.ds(lo, RSZ)])
        pltpu.sync_copy(accMx.at[pl.ds(0, RSZ)], mx_hbm.at[pl.ds(lo, RSZ)])
        pltpu.sync_copy(accMn.at[pl.ds(0, RSZ)], mn_hbm.at[pl.ds(lo, RSZ)])
        pltpu.sync_copy(accC.at[pl.ds(0, RSZ)], cnt_hbm.at[pl.ds(lo, RSZ)])
        return 0

    lax.fori_loop(0, 2, do_range, 0)


def _sc_stats(B, C, eids, srcs, dsts, cnts):
    stat = jax.ShapeDtypeStruct((NP, H), jnp.float32)
    k = pl.kernel(
        _stats_body,
        compiler_params=pltpu.CompilerParams(needs_layout_passes=False),
        out_type=[stat] * 4 + [jax.ShapeDtypeStruct((NP, 16), jnp.float32)],
        mesh=_sc_mesh(),
        scratch_types=[
            pltpu.VMEM((GCH,), jnp.int32),
            pltpu.VMEM((GCH,), jnp.int32),
            pltpu.VMEM((GCH + 16,), jnp.int32),
            pltpu.VMEM((GCH, H), jnp.float32),
            pltpu.VMEM((GCH, H), jnp.float32),
            pltpu.VMEM((GCH,), jnp.int32),
            pltpu.VMEM((GCH,), jnp.int32),
            pltpu.VMEM((GCH + 16,), jnp.int32),
            pltpu.VMEM((GCH, H), jnp.float32),
            pltpu.VMEM((GCH, H), jnp.float32),
            pltpu.VMEM((RSZ + 1, H), jnp.float32),
            pltpu.VMEM((RSZ + 1, H), jnp.float32),
            pltpu.VMEM((RSZ + 1, H), jnp.float32),
            pltpu.VMEM((RSZ + 1, H), jnp.float32),
            pltpu.VMEM((RSZ + 1, 16), jnp.float32),
            pltpu.VMEM((16,), jnp.int32),
            pltpu.SemaphoreType.DMA,
            pltpu.SemaphoreType.DMA,
            pltpu.SemaphoreType.DMA,
            pltpu.SemaphoreType.DMA,
        ],
    )
    return k(B, C, eids, srcs, dsts, cnts)


def kernel(x, edge_index, edge_attr, node_W, node_b, edge_W, edge_b,
           pre1_W, pre1_b, post1_W, post1_b, lin1_W, lin1_b, ln1_g, ln1_bt,
           pre2_W, pre2_b, post2_W, post2_b, lin2_W, lin2_b, ln2_g, ln2_bt):
    src = edge_index[0]
    dst = edge_index[1]
    Wd1, Ws1, We1 = pre1_W[:H], pre1_W[H:2 * H], pre1_W[2 * H:]
    Wd2, Ws2, We2 = pre2_W[:H], pre2_W[H:2 * H], pre2_W[2 * H:]

    h, A1, B1 = _node_in(x, node_W, node_b, Wd1, Ws1)
    C1, C2 = _edges(edge_attr, edge_W, edge_b, We1, pre1_b, We2, pre2_b)
    eids, srcs, dsts, cnts = _sc_prep(src, dst)

    S1, Q1, Mx1, Mn1, CNT1 = _sc_stats(B1, C1, eids, srcs, dsts, cnts)
    h1, A2, B2 = _post(h, A1, S1[:N], Q1[:N], Mx1[:N], Mn1[:N], CNT1[:N],
                       post1_W, post1_b, lin1_W, lin1_b, ln1_g, ln1_bt,
                       Wd2, Ws2)

    S2, Q2, Mx2, Mn2, CNT2 = _sc_stats(B2, C2, eids, srcs, dsts, cnts)
    h2, _, _ = _post(h1, A2, S2[:N], Q2[:N], Mx2[:N], Mn2[:N], CNT2[:N],
                     post2_W, post2_b, lin2_W, lin2_b, ln2_g, ln2_bt,
                     Wd2, Ws2)
    return h2
